# Initial kernel scaffold; baseline (speedup 1.0000x reference)
import jax
import jax.numpy as jnp
from jax.experimental import pallas as pl


def kernel(x, edge_attr, W_i, b_i, W_h, b_h, W_o, b_o, edge_index, rev_edge_index):
    raise NotImplementedError("write your pallas kernel here")


# speedup vs baseline: 1.3857x; 1.3857x over previous
import functools

import jax
import jax.numpy as jnp
from jax import lax
from jax.experimental import pallas as pl
from jax.experimental.pallas import tpu as pltpu
from jax.experimental.pallas import tpu_sc as plsc

NC = 2
NS = 16
NW = NC * NS


def _sc_mesh():
    return plsc.VectorSubcoreMesh(
        core_axis_name="c", subcore_axis_name="s", num_cores=NC, num_subcores=NS
    )


def _sc_gather(table, idx, chunk=80):
    b, = idx.shape
    _, dm = table.shape
    b_per_w = b // NW
    nch = b_per_w // chunk

    @functools.partial(
        pl.kernel,
        mesh=_sc_mesh(),
        out_type=jax.ShapeDtypeStruct((b, dm), jnp.float32),
        scratch_types=[
            pltpu.VMEM((chunk,), jnp.int32),
            pltpu.VMEM((chunk, dm), jnp.float32),
            pltpu.SemaphoreType.DMA,
        ],
    )
    def k(table_hbm, idx_hbm, out_hbm, idx_v, rows_v, sem):
        wid = lax.axis_index("s") * NC + lax.axis_index("c")
        base = wid * b_per_w

        def body(c, carry):
            off = pl.multiple_of(base + c * chunk, 8)
            pltpu.sync_copy(idx_hbm.at[pl.ds(off, chunk)], idx_v)
            pltpu.async_copy(table_hbm.at[idx_v], rows_v, sem).wait()
            pltpu.sync_copy(rows_v, out_hbm.at[pl.ds(off, chunk)])
            return carry

        lax.fori_loop(0, nch, body, 0)

    return k(table, idx)


def _sc_scatter_add(vals, dst, zeros, chunk=80):
    b, dm = vals.shape
    n = zeros.shape[0]
    b_per_w = b // NW
    nch = b_per_w // chunk

    @functools.partial(
        pl.kernel,
        mesh=_sc_mesh(),
        out_type=jax.ShapeDtypeStruct((NC, n, dm), jnp.float32),
        scratch_types=[
            pltpu.VMEM((chunk,), jnp.int32),
            pltpu.VMEM((chunk, dm), jnp.float32),
            pltpu.VMEM_SHARED((n, dm), jnp.float32),
        ],
    )
    def k(vals_hbm, dst_hbm, zeros_hbm, out_hbm, idx_v, rows_v, acc_sh):
        cid = lax.axis_index("c")
        sid = lax.axis_index("s")
        wid = sid * NC + cid

        @pl.when(sid == 0)
        def _():
            pltpu.sync_copy(zeros_hbm, acc_sh)

        plsc.subcore_barrier()
        base = wid * b_per_w

        def body(c, carry):
            off = pl.multiple_of(base + c * chunk, 8)
            pltpu.sync_copy(dst_hbm.at[pl.ds(off, chunk)], idx_v)
            pltpu.sync_copy(vals_hbm.at[pl.ds(off, chunk)], rows_v)
            pltpu.sync_copy(rows_v, acc_sh.at[idx_v], add=True)
            return carry

        lax.fori_loop(0, nch, body, 0)
        plsc.subcore_barrier()

        @pl.when(sid == 0)
        def _():
            pltpu.sync_copy(acc_sh, out_hbm.at[cid])

    return k(vals, dst, zeros)


def _row_specs(be, dm=128):
    return pl.BlockSpec((be, dm), lambda i: (i, 0))


def _const_spec(shape):
    return pl.BlockSpec(shape, lambda i: (0,) * len(shape))


def _tc_project(a, w_t, bn=1000):
    n, kdim = a.shape
    dm = w_t.shape[1]

    def body(a_ref, w_ref, o_ref):
        o_ref[...] = jnp.dot(a_ref[...], w_ref[...],
                             preferred_element_type=jnp.float32)

    return pl.pallas_call(
        body,
        grid=(n // bn,),
        in_specs=[_row_specs(bn, kdim), _const_spec((kdim, dm))],
        out_specs=_row_specs(bn, dm),
        out_shape=jax.ShapeDtypeStruct((n, dm), jnp.float32),
        compiler_params=pltpu.CompilerParams(dimension_semantics=("parallel",)),
    )(a, w_t)


def _tc_combine_project(p0, p1, w_t, bn=1000):
    n, kdim = p0.shape
    dm = w_t.shape[1]

    def body(p0_ref, p1_ref, w_ref, o_ref):
        o_ref[...] = jnp.dot(p0_ref[...] + p1_ref[...], w_ref[...],
                             preferred_element_type=jnp.float32)

    return pl.pallas_call(
        body,
        grid=(n // bn,),
        in_specs=[_row_specs(bn, kdim), _row_specs(bn, kdim),
                  _const_spec((kdim, dm))],
        out_specs=_row_specs(bn, dm),
        out_shape=jax.ShapeDtypeStruct((n, dm), jnp.float32),
        compiler_params=pltpu.CompilerParams(dimension_semantics=("parallel",)),
    )(p0, p1, w_t)


def _tc_h0(psrc, ea, we_t, b_i, be=512):
    e = psrc.shape[0]
    de = ea.shape[1]

    def body(ps_ref, ea_ref, w_ref, b_ref, h0_ref, ht_ref):
        h0 = ps_ref[...] + jnp.dot(ea_ref[...], w_ref[...],
                                   preferred_element_type=jnp.float32) + b_ref[...]
        h0_ref[...] = h0
        ht_ref[...] = jnp.maximum(h0, 0.0)

    return pl.pallas_call(
        body,
        grid=(e // be,),
        in_specs=[_row_specs(be), pl.BlockSpec((be, de), lambda i: (i, 0)),
                  _const_spec((de, 128)), _const_spec((1, 128))],
        out_specs=[_row_specs(be), _row_specs(be)],
        out_shape=[jax.ShapeDtypeStruct((e, 128), jnp.float32)] * 2,
        compiler_params=pltpu.CompilerParams(dimension_semantics=("parallel",)),
    )(psrc, ea, we_t, b_i)


def _tc_step(h0, qs, hr, wh_t, b_h, be=512):
    e = h0.shape[0]

    def body(h0_ref, qs_ref, hr_ref, w_ref, b_ref, o_ref):
        m = jnp.dot(hr_ref[...], w_ref[...], preferred_element_type=jnp.float32)
        o_ref[...] = jnp.maximum(h0_ref[...] + qs_ref[...] - m + b_ref[...], 0.0)

    return pl.pallas_call(
        body,
        grid=(e // be,),
        in_specs=[_row_specs(be), _row_specs(be), _row_specs(be),
                  _const_spec((128, 128)), _const_spec((1, 128))],
        out_specs=_row_specs(be),
        out_shape=jax.ShapeDtypeStruct((e, 128), jnp.float32),
        compiler_params=pltpu.CompilerParams(dimension_semantics=("parallel",)),
    )(h0, qs, hr, wh_t, b_h)


def _tc_out(x, p0, p1, wox_t, wom_t, b_o, bn=1000):
    n = x.shape[0]

    def body(x_ref, p0_ref, p1_ref, wx_ref, wm_ref, b_ref, o_ref):
        agg = p0_ref[...] + p1_ref[...]
        xv = x_ref[...]
        m = jnp.where(jnp.sum(agg, axis=1, keepdims=True) == 0.0, xv, agg)
        acc = jnp.dot(xv, wx_ref[...], preferred_element_type=jnp.float32)
        acc += jnp.dot(m, wm_ref[...], preferred_element_type=jnp.float32)
        o_ref[...] = jnp.maximum(acc + b_ref[...], 0.0)

    return pl.pallas_call(
        body,
        grid=(n // bn,),
        in_specs=[_row_specs(bn), _row_specs(bn), _row_specs(bn),
                  _const_spec((128, 128)), _const_spec((128, 128)),
                  _const_spec((1, 128))],
        out_specs=_row_specs(bn),
        out_shape=jax.ShapeDtypeStruct((n, 128), jnp.float32),
        compiler_params=pltpu.CompilerParams(dimension_semantics=("parallel",)),
    )(x, p0, p1, wox_t, wom_t, b_o)


def kernel(x, edge_attr, W_i, b_i, W_h, b_h, W_o, b_o, edge_index, rev_edge_index):
    n, d = x.shape
    hid = W_h.shape[0]
    src = edge_index[0]
    dst = edge_index[1]

    wx_t = W_i[:, :d].T
    we_t = W_i[:, d:].T
    wh_t = W_h.T
    wox_t = W_o[:, :d].T
    wom_t = W_o[:, d:].T
    b_i2 = b_i.reshape(1, hid)
    b_h2 = b_h.reshape(1, hid)
    b_o2 = b_o.reshape(1, hid)
    zeros_n = jnp.zeros((n, hid), jnp.float32)

    p = _tc_project(x, wx_t)
    psrc = _sc_gather(p, src)
    h0, ht = _tc_h0(psrc, edge_attr, we_t, b_i2)

    for _ in range(2):
        parts = _sc_scatter_add(ht, dst, zeros_n)
        q = _tc_combine_project(parts[0], parts[1], wh_t)
        qs = _sc_gather(q, src)
        hr = _sc_gather(ht, rev_edge_index)
        ht = _tc_step(h0, qs, hr, wh_t, b_h2)

    parts = _sc_scatter_add(ht, dst, zeros_n)
    return _tc_out(x, parts[0], parts[1], wox_t, wom_t, b_o2)


# speedup vs baseline: 1.6626x; 1.1998x over previous
import functools

import jax
import jax.numpy as jnp
from jax import lax
from jax.experimental import pallas as pl
from jax.experimental.pallas import tpu as pltpu
from jax.experimental.pallas import tpu_sc as plsc

NC = 2
NS = 16
NW = NC * NS


def _sc_mesh():
    return plsc.VectorSubcoreMesh(
        core_axis_name="c", subcore_axis_name="s", num_cores=NC, num_subcores=NS
    )


def _sc_gather(table, idx, chunk=400):
    b, = idx.shape
    _, dm = table.shape
    b_per_w = b // NW
    nch = b_per_w // chunk

    @functools.partial(
        pl.kernel,
        mesh=_sc_mesh(),
        out_type=jax.ShapeDtypeStruct((b, dm), jnp.float32),
        scratch_types=[
            pltpu.VMEM((chunk,), jnp.int32),
            pltpu.VMEM((chunk, dm), jnp.float32),
            pltpu.SemaphoreType.DMA,
        ],
    )
    def k(table_hbm, idx_hbm, out_hbm, idx_v, rows_v, sem):
        wid = lax.axis_index("s") * NC + lax.axis_index("c")
        base = wid * b_per_w

        def body(c, carry):
            off = pl.multiple_of(base + c * chunk, 8)
            pltpu.sync_copy(idx_hbm.at[pl.ds(off, chunk)], idx_v)
            pltpu.async_copy(table_hbm.at[idx_v], rows_v, sem).wait()
            pltpu.sync_copy(rows_v, out_hbm.at[pl.ds(off, chunk)])
            return carry

        lax.fori_loop(0, nch, body, 0)

    return k(table, idx)


def _sc_scatter_add(vals, dst, zeros, chunk=80):
    b, dm = vals.shape
    n = zeros.shape[0]
    b_per_w = b // NW
    nch = b_per_w // chunk

    @functools.partial(
        pl.kernel,
        mesh=_sc_mesh(),
        out_type=jax.ShapeDtypeStruct((NC, n, dm), jnp.float32),
        scratch_types=[
            pltpu.VMEM((chunk,), jnp.int32),
            pltpu.VMEM((chunk, dm), jnp.float32),
            pltpu.VMEM_SHARED((n, dm), jnp.float32),
        ],
    )
    def k(vals_hbm, dst_hbm, zeros_hbm, out_hbm, idx_v, rows_v, acc_sh):
        cid = lax.axis_index("c")
        sid = lax.axis_index("s")
        wid = sid * NC + cid

        @pl.when(sid == 0)
        def _():
            pltpu.sync_copy(zeros_hbm, acc_sh)

        plsc.subcore_barrier()
        base = wid * b_per_w

        def body(c, carry):
            off = pl.multiple_of(base + c * chunk, 8)
            pltpu.sync_copy(dst_hbm.at[pl.ds(off, chunk)], idx_v)
            pltpu.sync_copy(vals_hbm.at[pl.ds(off, chunk)], rows_v)
            pltpu.sync_copy(rows_v, acc_sh.at[idx_v], add=True)
            return carry

        lax.fori_loop(0, nch, body, 0)
        plsc.subcore_barrier()

        @pl.when(sid == 0)
        def _():
            pltpu.sync_copy(acc_sh, out_hbm.at[cid])

    return k(vals, dst, zeros)


def _row_specs(be, dm=128):
    return pl.BlockSpec((be, dm), lambda i: (i, 0))


def _const_spec(shape):
    return pl.BlockSpec(shape, lambda i: (0,) * len(shape))


def _tc_project(a, w_t, bn=1000):
    n, kdim = a.shape
    dm = w_t.shape[1]

    def body(a_ref, w_ref, o_ref):
        o_ref[...] = jnp.dot(a_ref[...], w_ref[...],
                             preferred_element_type=jnp.float32)

    return pl.pallas_call(
        body,
        grid=(n // bn,),
        in_specs=[_row_specs(bn, kdim), _const_spec((kdim, dm))],
        out_specs=_row_specs(bn, dm),
        out_shape=jax.ShapeDtypeStruct((n, dm), jnp.float32),
        compiler_params=pltpu.CompilerParams(dimension_semantics=("parallel",)),
    )(a, w_t)


def _tc_combine_project(p0, p1, w_t, bn=1000):
    n, kdim = p0.shape
    dm = w_t.shape[1]

    def body(p0_ref, p1_ref, w_ref, o_ref):
        o_ref[...] = jnp.dot(p0_ref[...] + p1_ref[...], w_ref[...],
                             preferred_element_type=jnp.float32)

    return pl.pallas_call(
        body,
        grid=(n // bn,),
        in_specs=[_row_specs(bn, kdim), _row_specs(bn, kdim),
                  _const_spec((kdim, dm))],
        out_specs=_row_specs(bn, dm),
        out_shape=jax.ShapeDtypeStruct((n, dm), jnp.float32),
        compiler_params=pltpu.CompilerParams(dimension_semantics=("parallel",)),
    )(p0, p1, w_t)


def _tc_h0(psrc, ea, we_t, b_i, be=512):
    e = psrc.shape[0]
    de = ea.shape[1]

    def body(ps_ref, ea_ref, w_ref, b_ref, h0_ref, ht_ref):
        h0 = ps_ref[...] + jnp.dot(ea_ref[...], w_ref[...],
                                   preferred_element_type=jnp.float32) + b_ref[...]
        h0_ref[...] = h0
        ht_ref[...] = jnp.maximum(h0, 0.0)

    return pl.pallas_call(
        body,
        grid=(e // be,),
        in_specs=[_row_specs(be), pl.BlockSpec((be, de), lambda i: (i, 0)),
                  _const_spec((de, 128)), _const_spec((1, 128))],
        out_specs=[_row_specs(be), _row_specs(be)],
        out_shape=[jax.ShapeDtypeStruct((e, 128), jnp.float32)] * 2,
        compiler_params=pltpu.CompilerParams(dimension_semantics=("parallel",)),
    )(psrc, ea, we_t, b_i)


def _tc_step(h0, qs, hr, wh_t, b_h, be=512):
    e = h0.shape[0]

    def body(h0_ref, qs_ref, hr_ref, w_ref, b_ref, o_ref):
        m = jnp.dot(hr_ref[...], w_ref[...], preferred_element_type=jnp.float32)
        o_ref[...] = jnp.maximum(h0_ref[...] + qs_ref[...] - m + b_ref[...], 0.0)

    return pl.pallas_call(
        body,
        grid=(e // be,),
        in_specs=[_row_specs(be), _row_specs(be), _row_specs(be),
                  _const_spec((128, 128)), _const_spec((1, 128))],
        out_specs=_row_specs(be),
        out_shape=jax.ShapeDtypeStruct((e, 128), jnp.float32),
        compiler_params=pltpu.CompilerParams(dimension_semantics=("parallel",)),
    )(h0, qs, hr, wh_t, b_h)


def _tc_out(x, p0, p1, wox_t, wom_t, b_o, bn=1000):
    n = x.shape[0]

    def body(x_ref, p0_ref, p1_ref, wx_ref, wm_ref, b_ref, o_ref):
        agg = p0_ref[...] + p1_ref[...]
        xv = x_ref[...]
        m = jnp.where(jnp.sum(agg, axis=1, keepdims=True) == 0.0, xv, agg)
        acc = jnp.dot(xv, wx_ref[...], preferred_element_type=jnp.float32)
        acc += jnp.dot(m, wm_ref[...], preferred_element_type=jnp.float32)
        o_ref[...] = jnp.maximum(acc + b_ref[...], 0.0)

    return pl.pallas_call(
        body,
        grid=(n // bn,),
        in_specs=[_row_specs(bn), _row_specs(bn), _row_specs(bn),
                  _const_spec((128, 128)), _const_spec((128, 128)),
                  _const_spec((1, 128))],
        out_specs=_row_specs(bn),
        out_shape=jax.ShapeDtypeStruct((n, 128), jnp.float32),
        compiler_params=pltpu.CompilerParams(dimension_semantics=("parallel",)),
    )(x, p0, p1, wox_t, wom_t, b_o)


def kernel(x, edge_attr, W_i, b_i, W_h, b_h, W_o, b_o, edge_index, rev_edge_index):
    n, d = x.shape
    hid = W_h.shape[0]
    src = edge_index[0]
    dst = edge_index[1]

    wx_t = W_i[:, :d].T
    we_t = W_i[:, d:].T
    wh_t = W_h.T
    wox_t = W_o[:, :d].T
    wom_t = W_o[:, d:].T
    b_i2 = b_i.reshape(1, hid)
    b_h2 = b_h.reshape(1, hid)
    b_o2 = b_o.reshape(1, hid)
    zeros_n = jnp.zeros((n, hid), jnp.float32)

    p = _tc_project(x, wx_t)
    psrc = _sc_gather(p, src)
    h0, ht = _tc_h0(psrc, edge_attr, we_t, b_i2)

    for _ in range(2):
        parts = _sc_scatter_add(ht, dst, zeros_n)
        q = _tc_combine_project(parts[0], parts[1], wh_t)
        qs = _sc_gather(q, src)
        hr = _sc_gather(ht, rev_edge_index)
        ht = _tc_step(h0, qs, hr, wh_t, b_h2)

    parts = _sc_scatter_add(ht, dst, zeros_n)
    return _tc_out(x, parts[0], parts[1], wox_t, wom_t, b_o2)


# speedup vs baseline: 1.9027x; 1.1445x over previous
import functools

import jax
import jax.numpy as jnp
from jax import lax
from jax.experimental import pallas as pl
from jax.experimental.pallas import tpu as pltpu
from jax.experimental.pallas import tpu_sc as plsc

NC = 2
NS = 16
NW = NC * NS


def _sc_mesh():
    return plsc.VectorSubcoreMesh(
        core_axis_name="c", subcore_axis_name="s", num_cores=NC, num_subcores=NS
    )


def _sc_gather(table, idx, chunk=200):
    b, = idx.shape
    _, dm = table.shape
    b_per_w = b // NW
    nch = b_per_w // chunk
    npair = nch // 2
    assert nch % 2 == 0

    @functools.partial(
        pl.kernel,
        mesh=_sc_mesh(),
        out_type=jax.ShapeDtypeStruct((b, dm), jnp.float32),
        scratch_types=[
            pltpu.VMEM((chunk,), jnp.int32),
            pltpu.VMEM((chunk,), jnp.int32),
            pltpu.VMEM((chunk, dm), jnp.float32),
            pltpu.VMEM((chunk, dm), jnp.float32),
            pltpu.SemaphoreType.DMA,
            pltpu.SemaphoreType.DMA,
            pltpu.SemaphoreType.DMA,
            pltpu.SemaphoreType.DMA,
            pltpu.SemaphoreType.DMA,
        ],
    )
    def k(table_hbm, idx_hbm, out_hbm, idx_a, idx_b, rows_a, rows_b,
          sem_ia, sem_ib, sem_g, sem_sa, sem_sb):
        wid = lax.axis_index("s") * NC + lax.axis_index("c")
        base = wid * b_per_w

        def idx_cp(c, buf, sem):
            off = pl.multiple_of(base + c * chunk, 8)
            return pltpu.make_async_copy(idx_hbm.at[pl.ds(off, chunk)], buf, sem)

        def store_cp(c, buf, sem):
            off = pl.multiple_of(base + c * chunk, 8)
            return pltpu.make_async_copy(buf, out_hbm.at[pl.ds(off, chunk)], sem)

        idx_cp(0, idx_a, sem_ia).start()

        def body(j, carry):
            c0 = 2 * j
            c1 = c0 + 1
            idx_cp(c0, idx_a, sem_ia).wait()
            idx_cp(c1, idx_b, sem_ib).start()

            @pl.when(j >= 1)
            def _():
                store_cp(c0 - 2, rows_a, sem_sa).wait()

            g = pltpu.make_async_copy(table_hbm.at[idx_a], rows_a, sem_g)
            g.start()
            g.wait()
            store_cp(c0, rows_a, sem_sa).start()
            idx_cp(c1, idx_b, sem_ib).wait()

            @pl.when(j + 1 < npair)
            def _():
                idx_cp(c0 + 2, idx_a, sem_ia).start()

            @pl.when(j >= 1)
            def _():
                store_cp(c1 - 2, rows_b, sem_sb).wait()

            g2 = pltpu.make_async_copy(table_hbm.at[idx_b], rows_b, sem_g)
            g2.start()
            g2.wait()
            store_cp(c1, rows_b, sem_sb).start()
            return carry

        lax.fori_loop(0, npair, body, 0)
        store_cp(nch - 2, rows_a, sem_sa).wait()
        store_cp(nch - 1, rows_b, sem_sb).wait()

    return k(table, idx)


def _sc_scatter_add(vals, dst, zeros, chunk=80):
    b, dm = vals.shape
    n = zeros.shape[0]
    b_per_w = b // NW
    nch = b_per_w // chunk
    npair = nch // 2
    tail = nch % 2 == 1

    @functools.partial(
        pl.kernel,
        mesh=_sc_mesh(),
        out_type=jax.ShapeDtypeStruct((NC, n, dm), jnp.float32),
        scratch_types=[
            pltpu.VMEM((chunk,), jnp.int32),
            pltpu.VMEM((chunk,), jnp.int32),
            pltpu.VMEM((chunk, dm), jnp.float32),
            pltpu.VMEM((chunk, dm), jnp.float32),
            pltpu.VMEM_SHARED((n, dm), jnp.float32),
            pltpu.SemaphoreType.DMA,
            pltpu.SemaphoreType.DMA,
            pltpu.SemaphoreType.DMA,
            pltpu.SemaphoreType.DMA,
            pltpu.SemaphoreType.DMA,
            pltpu.SemaphoreType.DMA,
        ],
    )
    def k(vals_hbm, dst_hbm, zeros_hbm, out_hbm, idx_a, idx_b, rows_a, rows_b,
          acc_sh, sem_ia, sem_ib, sem_va, sem_vb, sem_sa, sem_sb):
        cid = lax.axis_index("c")
        sid = lax.axis_index("s")
        wid = sid * NC + cid
        base = wid * b_per_w

        def ld_cp(c, hbm, buf, sem):
            off = pl.multiple_of(base + c * chunk, 8)
            return pltpu.make_async_copy(hbm.at[pl.ds(off, chunk)], buf, sem)

        def sc_cp(ibuf, rbuf, sem):
            return pltpu.make_async_copy(rbuf, acc_sh.at[ibuf], sem)

        @pl.when(sid == 0)
        def _():
            pltpu.sync_copy(zeros_hbm, acc_sh)

        plsc.subcore_barrier()
        ld_cp(0, dst_hbm, idx_a, sem_ia).start()
        ld_cp(0, vals_hbm, rows_a, sem_va).start()

        def body(j, carry):
            c0 = 2 * j
            c1 = c0 + 1
            ld_cp(c0, dst_hbm, idx_a, sem_ia).wait()
            ld_cp(c0, vals_hbm, rows_a, sem_va).wait()

            @pl.when(j >= 1)
            def _():
                sc_cp(idx_b, rows_b, sem_sb).wait()

            ld_cp(c1, dst_hbm, idx_b, sem_ib).start()
            ld_cp(c1, vals_hbm, rows_b, sem_vb).start()
            sc_cp(idx_a, rows_a, sem_sa).start(add=True)
            ld_cp(c1, dst_hbm, idx_b, sem_ib).wait()
            ld_cp(c1, vals_hbm, rows_b, sem_vb).wait()
            sc_cp(idx_a, rows_a, sem_sa).wait()

            @pl.when(c0 + 2 < nch)
            def _():
                ld_cp(c0 + 2, dst_hbm, idx_a, sem_ia).start()
                ld_cp(c0 + 2, vals_hbm, rows_a, sem_va).start()

            sc_cp(idx_b, rows_b, sem_sb).start(add=True)
            return carry

        lax.fori_loop(0, npair, body, 0)
        sc_cp(idx_b, rows_b, sem_sb).wait()
        if tail:
            c = nch - 1
            ld_cp(c, dst_hbm, idx_a, sem_ia).wait()
            ld_cp(c, vals_hbm, rows_a, sem_va).wait()
            scd = sc_cp(idx_a, rows_a, sem_sa)
            scd.start(add=True)
            scd.wait()
        plsc.subcore_barrier()

        @pl.when(sid == 0)
        def _():
            pltpu.sync_copy(acc_sh, out_hbm.at[cid])

    return k(vals, dst, zeros)


def _row_specs(be, dm=128):
    return pl.BlockSpec((be, dm), lambda i: (i, 0))


def _const_spec(shape):
    return pl.BlockSpec(shape, lambda i: (0,) * len(shape))


def _tc_project(a, w_t, bn=1000):
    n, kdim = a.shape
    dm = w_t.shape[1]

    def body(a_ref, w_ref, o_ref):
        o_ref[...] = jnp.dot(a_ref[...], w_ref[...],
                             preferred_element_type=jnp.float32)

    return pl.pallas_call(
        body,
        grid=(n // bn,),
        in_specs=[_row_specs(bn, kdim), _const_spec((kdim, dm))],
        out_specs=_row_specs(bn, dm),
        out_shape=jax.ShapeDtypeStruct((n, dm), jnp.float32),
        compiler_params=pltpu.CompilerParams(dimension_semantics=("parallel",)),
    )(a, w_t)


def _tc_combine_project(p0, p1, w_t, bn=1000):
    n, kdim = p0.shape
    dm = w_t.shape[1]

    def body(p0_ref, p1_ref, w_ref, o_ref):
        o_ref[...] = jnp.dot(p0_ref[...] + p1_ref[...], w_ref[...],
                             preferred_element_type=jnp.float32)

    return pl.pallas_call(
        body,
        grid=(n // bn,),
        in_specs=[_row_specs(bn, kdim), _row_specs(bn, kdim),
                  _const_spec((kdim, dm))],
        out_specs=_row_specs(bn, dm),
        out_shape=jax.ShapeDtypeStruct((n, dm), jnp.float32),
        compiler_params=pltpu.CompilerParams(dimension_semantics=("parallel",)),
    )(p0, p1, w_t)


def _tc_h0(psrc, ea, we_t, b_i, be=512):
    e = psrc.shape[0]
    de = ea.shape[1]

    def body(ps_ref, ea_ref, w_ref, b_ref, h0_ref, ht_ref):
        h0 = ps_ref[...] + jnp.dot(ea_ref[...], w_ref[...],
                                   preferred_element_type=jnp.float32) + b_ref[...]
        h0_ref[...] = h0
        ht_ref[...] = jnp.maximum(h0, 0.0)

    return pl.pallas_call(
        body,
        grid=(e // be,),
        in_specs=[_row_specs(be), pl.BlockSpec((be, de), lambda i: (i, 0)),
                  _const_spec((de, 128)), _const_spec((1, 128))],
        out_specs=[_row_specs(be), _row_specs(be)],
        out_shape=[jax.ShapeDtypeStruct((e, 128), jnp.float32)] * 2,
        compiler_params=pltpu.CompilerParams(dimension_semantics=("parallel",)),
    )(psrc, ea, we_t, b_i)


def _tc_step(h0, qs, hr, wh_t, b_h, be=512):
    e = h0.shape[0]

    def body(h0_ref, qs_ref, hr_ref, w_ref, b_ref, o_ref):
        m = jnp.dot(hr_ref[...], w_ref[...], preferred_element_type=jnp.float32)
        o_ref[...] = jnp.maximum(h0_ref[...] + qs_ref[...] - m + b_ref[...], 0.0)

    return pl.pallas_call(
        body,
        grid=(e // be,),
        in_specs=[_row_specs(be), _row_specs(be), _row_specs(be),
                  _const_spec((128, 128)), _const_spec((1, 128))],
        out_specs=_row_specs(be),
        out_shape=jax.ShapeDtypeStruct((e, 128), jnp.float32),
        compiler_params=pltpu.CompilerParams(dimension_semantics=("parallel",)),
    )(h0, qs, hr, wh_t, b_h)


def _tc_out(x, p0, p1, wox_t, wom_t, b_o, bn=1000):
    n = x.shape[0]

    def body(x_ref, p0_ref, p1_ref, wx_ref, wm_ref, b_ref, o_ref):
        agg = p0_ref[...] + p1_ref[...]
        xv = x_ref[...]
        m = jnp.where(jnp.sum(agg, axis=1, keepdims=True) == 0.0, xv, agg)
        acc = jnp.dot(xv, wx_ref[...], preferred_element_type=jnp.float32)
        acc += jnp.dot(m, wm_ref[...], preferred_element_type=jnp.float32)
        o_ref[...] = jnp.maximum(acc + b_ref[...], 0.0)

    return pl.pallas_call(
        body,
        grid=(n // bn,),
        in_specs=[_row_specs(bn), _row_specs(bn), _row_specs(bn),
                  _const_spec((128, 128)), _const_spec((128, 128)),
                  _const_spec((1, 128))],
        out_specs=_row_specs(bn),
        out_shape=jax.ShapeDtypeStruct((n, 128), jnp.float32),
        compiler_params=pltpu.CompilerParams(dimension_semantics=("parallel",)),
    )(x, p0, p1, wox_t, wom_t, b_o)


def kernel(x, edge_attr, W_i, b_i, W_h, b_h, W_o, b_o, edge_index, rev_edge_index):
    n, d = x.shape
    hid = W_h.shape[0]
    src = edge_index[0]
    dst = edge_index[1]

    wx_t = W_i[:, :d].T
    we_t = W_i[:, d:].T
    wh_t = W_h.T
    wox_t = W_o[:, :d].T
    wom_t = W_o[:, d:].T
    b_i2 = b_i.reshape(1, hid)
    b_h2 = b_h.reshape(1, hid)
    b_o2 = b_o.reshape(1, hid)
    zeros_n = jnp.zeros((n, hid), jnp.float32)

    p = _tc_project(x, wx_t)
    psrc = _sc_gather(p, src)
    h0, ht = _tc_h0(psrc, edge_attr, we_t, b_i2)

    for _ in range(2):
        parts = _sc_scatter_add(ht, dst, zeros_n)
        q = _tc_combine_project(parts[0], parts[1], wh_t)
        qs = _sc_gather(q, src)
        hr = _sc_gather(ht, rev_edge_index)
        ht = _tc_step(h0, qs, hr, wh_t, b_h2)

    parts = _sc_scatter_add(ht, dst, zeros_n)
    return _tc_out(x, parts[0], parts[1], wox_t, wom_t, b_o2)
